# baseline (speedup 1.0000x reference)
import functools

import jax
import jax.numpy as jnp
from jax import lax
from jax.experimental import pallas as pl
from jax.experimental.pallas import tpu as pltpu
from jax.experimental.pallas import tpu_sc as plsc

N = 10000
D = 128
E = 320000
H = 8
GROUPS = 16
R = E // GROUPS
EPS = 1e-5


def _stage_a_body(x_ref, w1_ref, bias_ref, p_ref, q_ref):
    x = x_ref[...]
    p_ref[...] = (
        jnp.dot(x, w1_ref[0:D, :], preferred_element_type=jnp.float32)
        + bias_ref[...]
    )
    q_ref[...] = jnp.dot(x, w1_ref[D:2 * D, :],
                         preferred_element_type=jnp.float32)


def _stage_a(x, w1, bias):
    return pl.pallas_call(
        _stage_a_body,
        out_shape=[
            jax.ShapeDtypeStruct((N, H), jnp.float32),
            jax.ShapeDtypeStruct((N, H), jnp.float32),
        ],
    )(x, w1, bias)


_INFO = plsc.get_sparse_core_info()
_NC = _INFO.num_cores
_NS = _INFO.num_subcores
_NW = _NC * _NS
_EPW = E // _NW
_CHUNK = 128
_NFULL = _EPW // _CHUNK
_TAIL = _EPW - _NFULL * _CHUNK
_NBUF = 6
_NOUTER = _NFULL // _NBUF


def _stage_b_kernel(p_hbm, q_hbm, s_hbm, e_hbm, out1_hbm, out2_hbm,
                    idx_s, idx_e, *bufs):
    rows_s = bufs[0:_NBUF]
    rows_e = bufs[_NBUF:2 * _NBUF]
    sem_gs = bufs[2 * _NBUF:3 * _NBUF]
    sem_ge = bufs[3 * _NBUF:4 * _NBUF]
    sem_os = bufs[4 * _NBUF:5 * _NBUF]
    sem_oe = bufs[5 * _NBUF:6 * _NBUF]

    wid = lax.axis_index("s") * _NC + lax.axis_index("c")
    base = wid * _EPW

    def gather_pair(c, b):
        si = idx_s.at[pl.ds(c * _CHUNK, _CHUNK)]
        ei = idx_e.at[pl.ds(c * _CHUNK, _CHUNK)]
        pltpu.async_copy(p_hbm.at[si], rows_s[b], sem_gs[b])
        pltpu.async_copy(q_hbm.at[ei], rows_e[b], sem_ge[b])

    def wait_gather(c, b):
        si = idx_s.at[pl.ds(c * _CHUNK, _CHUNK)]
        ei = idx_e.at[pl.ds(c * _CHUNK, _CHUNK)]
        pltpu.make_async_copy(p_hbm.at[si], rows_s[b], sem_gs[b]).wait()
        pltpu.make_async_copy(q_hbm.at[ei], rows_e[b], sem_ge[b]).wait()

    def start_out(c, b):
        cb = base + c * _CHUNK
        pltpu.async_copy(rows_s[b], out1_hbm.at[pl.ds(cb, _CHUNK)], sem_os[b])
        pltpu.async_copy(rows_e[b], out2_hbm.at[pl.ds(cb, _CHUNK)], sem_oe[b])

    def wait_out(c, b):
        cb = base + c * _CHUNK
        pltpu.make_async_copy(
            rows_s[b], out1_hbm.at[pl.ds(cb, _CHUNK)], sem_os[b]).wait()
        pltpu.make_async_copy(
            rows_e[b], out2_hbm.at[pl.ds(cb, _CHUNK)], sem_oe[b]).wait()

    pltpu.sync_copy(s_hbm.at[pl.ds(base, _EPW)], idx_s)
    pltpu.sync_copy(e_hbm.at[pl.ds(base, _EPW)], idx_e)

    for b in range(4):
        gather_pair(jnp.int32(b), b)

    def outer(g, carry):
        for b in range(_NBUF):
            t = g * _NBUF + b
            wait_gather(t, b)
            start_out(t, b)
            tg = t + 4
            bg = (b + 4) % _NBUF

            @pl.when(tg < _NFULL)
            def _():
                @pl.when(t >= 2)
                def _():
                    wait_out(t - 2, bg)
                gather_pair(tg, bg)
        return carry

    lax.fori_loop(0, _NOUTER, outer, None)

    for k in range(_NBUF):
        c = _NFULL - _NBUF + k
        wait_out(jnp.int32(c), c % _NBUF)

    tb = base + _NFULL * _CHUNK
    si = idx_s.at[pl.ds(_NFULL * _CHUNK, _TAIL)]
    ei = idx_e.at[pl.ds(_NFULL * _CHUNK, _TAIL)]
    ts = rows_s[0].at[pl.ds(0, _TAIL), :]
    te = rows_e[0].at[pl.ds(0, _TAIL), :]
    pltpu.async_copy(p_hbm.at[si], ts, sem_gs[0])
    pltpu.async_copy(q_hbm.at[ei], te, sem_ge[0])
    pltpu.make_async_copy(p_hbm.at[si], ts, sem_gs[0]).wait()
    pltpu.make_async_copy(q_hbm.at[ei], te, sem_ge[0]).wait()
    pltpu.sync_copy(ts, out1_hbm.at[pl.ds(tb, _TAIL)])
    pltpu.sync_copy(te, out2_hbm.at[pl.ds(tb, _TAIL)])


def _stage_b(p_tab, q_tab, start, end):
    fn = functools.partial(
        pl.kernel,
        mesh=plsc.VectorSubcoreMesh(core_axis_name="c", subcore_axis_name="s"),
        compiler_params=pltpu.CompilerParams(use_tc_tiling_on_sc=False),
        out_type=[
            jax.ShapeDtypeStruct((E, H), jnp.float32),
            jax.ShapeDtypeStruct((E, H), jnp.float32),
        ],
        scratch_types=[
            pltpu.VMEM((_EPW,), jnp.int32),
            pltpu.VMEM((_EPW,), jnp.int32),
        ]
        + [pltpu.VMEM((_CHUNK, H), jnp.float32) for _ in range(2 * _NBUF)]
        + [pltpu.SemaphoreType.DMA for _ in range(4 * _NBUF)],
    )(_stage_b_kernel)
    return fn(p_tab, q_tab, start, end)


def _stage_c_body(s_ref, e_ref, w2_ref, w3_ref, w4_ref, vraw_ref, out_ref):
    f32 = jnp.float32
    ii = lax.broadcasted_iota(jnp.int32, (128, 128), 0)
    jj = lax.broadcasted_iota(jnp.int32, (128, 128), 1)
    bd1 = jnp.where((ii // H) == (jj // H), 1.0, 0.0).astype(f32)
    si = lax.broadcasted_iota(jnp.int32, (128, H), 0)
    sj = lax.broadcasted_iota(jnp.int32, (128, H), 1)
    s_mat = jnp.where((si % H) == sj, 1.0, 0.0).astype(f32)
    t8i = lax.broadcasted_iota(jnp.int32, (H, 128), 0)
    t8j = lax.broadcasted_iota(jnp.int32, (H, 128), 1)
    t8m = jnp.where((t8j % H) == t8i, 1.0, 0.0).astype(f32)

    def bd(w):
        tile = jnp.dot(jnp.dot(s_mat, w, preferred_element_type=f32), t8m,
                       preferred_element_type=f32)
        return tile * bd1

    w2bd = bd(w2_ref[...])
    w3bd = bd(w3_ref[...])
    ci = lax.broadcasted_iota(jnp.int32, (128, GROUPS), 0)
    cg = lax.broadcasted_iota(jnp.int32, (128, GROUPS), 1)
    colmask = jnp.where((ci // H) == cg, 1.0, 0.0).astype(f32)
    w4col = jnp.dot(s_mat, w4_ref[...], preferred_element_type=f32)
    c4 = w4col * colmask
    vt = jnp.dot(vraw_ref[...], t8m, preferred_element_type=f32)

    def ln_relu(z, g, be):
        m = jnp.dot(z, bd1, preferred_element_type=f32) * (1.0 / H)
        zc = z - m
        v = jnp.dot(zc * zc, bd1, preferred_element_type=f32) * (1.0 / H)
        z = zc / jnp.sqrt(v + EPS) * g + be
        return jnp.maximum(z, 0.0)

    z = s_ref[...] + e_ref[...]
    z = ln_relu(z, vt[0:1, :], vt[1:2, :])
    z = jnp.dot(z, w2bd, preferred_element_type=f32) + vt[2:3, :]
    z = ln_relu(z, vt[3:4, :], vt[4:5, :])
    z = jnp.dot(z, w3bd, preferred_element_type=f32) + vt[5:6, :]
    z = ln_relu(z, vt[6:7, :], vt[7:8, :])
    out_ref[...] = (
        jnp.dot(z, c4, preferred_element_type=f32) + vt[8:9, 0:GROUPS]
    )


def _stage_c(zs, ze, w2, w3, w4, vraw):
    rb = 2000
    grid = R // rb
    return pl.pallas_call(
        _stage_c_body,
        grid=(grid,),
        in_specs=[
            pl.BlockSpec((rb, 128), lambda i: (i, 0)),
            pl.BlockSpec((rb, 128), lambda i: (i, 0)),
            pl.BlockSpec((H, H), lambda i: (0, 0)),
            pl.BlockSpec((H, H), lambda i: (0, 0)),
            pl.BlockSpec((H, 1), lambda i: (0, 0)),
            pl.BlockSpec((9, H), lambda i: (0, 0)),
        ],
        out_specs=pl.BlockSpec((rb, GROUPS), lambda i: (i, 0)),
        out_shape=jax.ShapeDtypeStruct((R, GROUPS), jnp.float32),
    )(zs, ze, w2, w3, w4, vraw)


def kernel(x, edge_index, W1, b1, g1, be1, W2, b2, g2, be2, W3, b3, g3, be3,
           W4, b4):
    bias = b1[None, :]
    vraw = jnp.stack([g1, be1, b2, g2, be2, b3, g3, be3,
                      jnp.full((H,), b4[0], jnp.float32)])

    start = edge_index[0].astype(jnp.int32)
    end = edge_index[1].astype(jnp.int32)

    p_tab, q_tab = _stage_a(x, W1, bias)
    rows_s, rows_e = _stage_b(p_tab, q_tab, start, end)
    zs = rows_s.reshape(R, 128)
    ze = rows_e.reshape(R, 128)
    out16 = _stage_c(zs, ze, W2, W3, W4, vraw)
    return out16.reshape(E)


# speedup vs baseline: 1.0114x; 1.0114x over previous
import functools

import jax
import jax.numpy as jnp
from jax import lax
from jax.experimental import pallas as pl
from jax.experimental.pallas import tpu as pltpu
from jax.experimental.pallas import tpu_sc as plsc

N = 10000
D = 128
E = 320000
H = 8
GROUPS = 16
R = E // GROUPS
EPS = 1e-5


def _stage_a_body(x_ref, w1_ref, bias_ref, p_ref, q_ref):
    x = x_ref[...]
    p_ref[...] = (
        jnp.dot(x, w1_ref[0:D, :], preferred_element_type=jnp.float32)
        + bias_ref[...]
    )
    q_ref[...] = jnp.dot(x, w1_ref[D:2 * D, :],
                         preferred_element_type=jnp.float32)


def _stage_a(x, w1, bias):
    return pl.pallas_call(
        _stage_a_body,
        out_shape=[
            jax.ShapeDtypeStruct((N, H), jnp.float32),
            jax.ShapeDtypeStruct((N, H), jnp.float32),
        ],
    )(x, w1, bias)


_INFO = plsc.get_sparse_core_info()
_NC = _INFO.num_cores
_NS = _INFO.num_subcores
_NW = _NC * _NS
_EPW = E // _NW
_CHUNK = 128
_NFULL = _EPW // _CHUNK
_TAIL = _EPW - _NFULL * _CHUNK
_NBUF = 6
_NOUTER = _NFULL // _NBUF


def _stage_b_kernel(p_hbm, q_hbm, s_hbm, e_hbm, out1_hbm, out2_hbm,
                    idx_s, idx_e, *bufs):
    rows_s = bufs[0:_NBUF]
    rows_e = bufs[_NBUF:2 * _NBUF]
    sem_gs = bufs[2 * _NBUF:3 * _NBUF]
    sem_ge = bufs[3 * _NBUF:4 * _NBUF]
    sem_os = bufs[4 * _NBUF:5 * _NBUF]
    sem_oe = bufs[5 * _NBUF:6 * _NBUF]

    wid = lax.axis_index("s") * _NC + lax.axis_index("c")
    base = wid * _EPW

    def gather_pair(c, b):
        si = idx_s.at[pl.ds(c * _CHUNK, _CHUNK)]
        ei = idx_e.at[pl.ds(c * _CHUNK, _CHUNK)]
        pltpu.async_copy(p_hbm.at[si], rows_s[b], sem_gs[b])
        pltpu.async_copy(q_hbm.at[ei], rows_e[b], sem_ge[b])

    def wait_gather(c, b):
        si = idx_s.at[pl.ds(c * _CHUNK, _CHUNK)]
        ei = idx_e.at[pl.ds(c * _CHUNK, _CHUNK)]
        pltpu.make_async_copy(p_hbm.at[si], rows_s[b], sem_gs[b]).wait()
        pltpu.make_async_copy(q_hbm.at[ei], rows_e[b], sem_ge[b]).wait()

    def start_out(c, b):
        cb = base + c * _CHUNK
        pltpu.async_copy(rows_s[b], out1_hbm.at[pl.ds(cb, _CHUNK)], sem_os[b])
        pltpu.async_copy(rows_e[b], out2_hbm.at[pl.ds(cb, _CHUNK)], sem_oe[b])

    def wait_out(c, b):
        cb = base + c * _CHUNK
        pltpu.make_async_copy(
            rows_s[b], out1_hbm.at[pl.ds(cb, _CHUNK)], sem_os[b]).wait()
        pltpu.make_async_copy(
            rows_e[b], out2_hbm.at[pl.ds(cb, _CHUNK)], sem_oe[b]).wait()

    pltpu.sync_copy(s_hbm.at[pl.ds(base, _EPW)], idx_s)
    pltpu.sync_copy(e_hbm.at[pl.ds(base, _EPW)], idx_e)

    for b in range(4):
        gather_pair(jnp.int32(b), b)

    def outer(g, carry):
        for b in range(_NBUF):
            t = g * _NBUF + b
            wait_gather(t, b)
            start_out(t, b)
            tg = t + 4
            bg = (b + 4) % _NBUF

            @pl.when(tg < _NFULL)
            def _():
                @pl.when(t >= 2)
                def _():
                    wait_out(t - 2, bg)
                gather_pair(tg, bg)
        return carry

    lax.fori_loop(0, _NOUTER, outer, None)

    for k in range(_NBUF):
        c = _NFULL - _NBUF + k
        wait_out(jnp.int32(c), c % _NBUF)

    tb = base + _NFULL * _CHUNK
    si = idx_s.at[pl.ds(_NFULL * _CHUNK, _TAIL)]
    ei = idx_e.at[pl.ds(_NFULL * _CHUNK, _TAIL)]
    ts = rows_s[0].at[pl.ds(0, _TAIL), :]
    te = rows_e[0].at[pl.ds(0, _TAIL), :]
    pltpu.async_copy(p_hbm.at[si], ts, sem_gs[0])
    pltpu.async_copy(q_hbm.at[ei], te, sem_ge[0])
    pltpu.make_async_copy(p_hbm.at[si], ts, sem_gs[0]).wait()
    pltpu.make_async_copy(q_hbm.at[ei], te, sem_ge[0]).wait()
    pltpu.sync_copy(ts, out1_hbm.at[pl.ds(tb, _TAIL)])
    pltpu.sync_copy(te, out2_hbm.at[pl.ds(tb, _TAIL)])


def _stage_b(p_tab, q_tab, start, end):
    fn = functools.partial(
        pl.kernel,
        mesh=plsc.VectorSubcoreMesh(core_axis_name="c", subcore_axis_name="s"),
        compiler_params=pltpu.CompilerParams(use_tc_tiling_on_sc=False),
        out_type=[
            jax.ShapeDtypeStruct((E, H), jnp.float32),
            jax.ShapeDtypeStruct((E, H), jnp.float32),
        ],
        scratch_types=[
            pltpu.VMEM((_EPW,), jnp.int32),
            pltpu.VMEM((_EPW,), jnp.int32),
        ]
        + [pltpu.VMEM((_CHUNK, H), jnp.float32) for _ in range(2 * _NBUF)]
        + [pltpu.SemaphoreType.DMA for _ in range(4 * _NBUF)],
    )(_stage_b_kernel)
    return fn(p_tab, q_tab, start, end)


def _stage_c_body(s_ref, e_ref, w2_ref, w3_ref, w4_ref, vraw_ref, out_ref):
    f32 = jnp.float32
    ii = lax.broadcasted_iota(jnp.int32, (128, 128), 0)
    jj = lax.broadcasted_iota(jnp.int32, (128, 128), 1)
    bd1 = jnp.where((ii // H) == (jj // H), 1.0, 0.0).astype(f32)
    si = lax.broadcasted_iota(jnp.int32, (128, H), 0)
    sj = lax.broadcasted_iota(jnp.int32, (128, H), 1)
    s_mat = jnp.where((si % H) == sj, 1.0, 0.0).astype(f32)
    t8i = lax.broadcasted_iota(jnp.int32, (H, 128), 0)
    t8j = lax.broadcasted_iota(jnp.int32, (H, 128), 1)
    t8m = jnp.where((t8j % H) == t8i, 1.0, 0.0).astype(f32)

    def bd(w):
        tile = jnp.dot(jnp.dot(s_mat, w, preferred_element_type=f32), t8m,
                       preferred_element_type=f32)
        return tile * bd1

    w2bd = bd(w2_ref[...])
    w3bd = bd(w3_ref[...])
    ci = lax.broadcasted_iota(jnp.int32, (128, GROUPS), 0)
    cg = lax.broadcasted_iota(jnp.int32, (128, GROUPS), 1)
    colmask = jnp.where((ci // H) == cg, 1.0, 0.0).astype(f32)
    w4col = jnp.dot(s_mat, w4_ref[...], preferred_element_type=f32)
    c4 = w4col * colmask
    vt = jnp.dot(vraw_ref[...], t8m, preferred_element_type=f32)

    def ln_relu(z, g, be):
        m = jnp.dot(z, bd1, preferred_element_type=f32) * (1.0 / H)
        zc = z - m
        v = jnp.dot(zc * zc, bd1, preferred_element_type=f32) * (1.0 / H)
        z = zc / jnp.sqrt(v + EPS) * g + be
        return jnp.maximum(z, 0.0)

    z = s_ref[...] + e_ref[...]
    z = ln_relu(z, vt[0:1, :], vt[1:2, :])
    z = jnp.dot(z, w2bd, preferred_element_type=f32) + vt[2:3, :]
    z = ln_relu(z, vt[3:4, :], vt[4:5, :])
    z = jnp.dot(z, w3bd, preferred_element_type=f32) + vt[5:6, :]
    z = ln_relu(z, vt[6:7, :], vt[7:8, :])
    out_ref[...] = (
        jnp.dot(z, c4, preferred_element_type=f32) + vt[8:9, 0:GROUPS]
    )


def _stage_c(zs, ze, w2, w3, w4, vraw):
    rb = 4000
    grid = R // rb
    return pl.pallas_call(
        _stage_c_body,
        grid=(grid,),
        in_specs=[
            pl.BlockSpec((rb, 128), lambda i: (i, 0)),
            pl.BlockSpec((rb, 128), lambda i: (i, 0)),
            pl.BlockSpec((H, H), lambda i: (0, 0)),
            pl.BlockSpec((H, H), lambda i: (0, 0)),
            pl.BlockSpec((H, 1), lambda i: (0, 0)),
            pl.BlockSpec((9, H), lambda i: (0, 0)),
        ],
        out_specs=pl.BlockSpec((rb, GROUPS), lambda i: (i, 0)),
        out_shape=jax.ShapeDtypeStruct((R, GROUPS), jnp.float32),
    )(zs, ze, w2, w3, w4, vraw)


def kernel(x, edge_index, W1, b1, g1, be1, W2, b2, g2, be2, W3, b3, g3, be3,
           W4, b4):
    bias = b1[None, :]
    vraw = jnp.stack([g1, be1, b2, g2, be2, b3, g3, be3,
                      jnp.full((H,), b4[0], jnp.float32)])

    start = edge_index[0].astype(jnp.int32)
    end = edge_index[1].astype(jnp.int32)

    p_tab, q_tab = _stage_a(x, W1, bias)
    rows_s, rows_e = _stage_b(p_tab, q_tab, start, end)
    zs = rows_s.reshape(R, 128)
    ze = rows_e.reshape(R, 128)
    out16 = _stage_c(zs, ze, W2, W3, W4, vraw)
    return out16.reshape(E)


# speedup vs baseline: 1.0218x; 1.0103x over previous
import functools

import jax
import jax.numpy as jnp
from jax import lax
from jax.experimental import pallas as pl
from jax.experimental.pallas import tpu as pltpu
from jax.experimental.pallas import tpu_sc as plsc

N = 10000
D = 128
E = 320000
H = 8
GROUPS = 16
R = E // GROUPS
EPS = 1e-5


def _stage_a_body(x_ref, w1_ref, bias_ref, p_ref, q_ref):
    x = x_ref[...]
    p_ref[...] = (
        jnp.dot(x, w1_ref[0:D, :], preferred_element_type=jnp.float32)
        + bias_ref[...]
    )
    q_ref[...] = jnp.dot(x, w1_ref[D:2 * D, :],
                         preferred_element_type=jnp.float32)


def _stage_a(x, w1, bias):
    return pl.pallas_call(
        _stage_a_body,
        out_shape=[
            jax.ShapeDtypeStruct((N, H), jnp.float32),
            jax.ShapeDtypeStruct((N, H), jnp.float32),
        ],
    )(x, w1, bias)


_INFO = plsc.get_sparse_core_info()
_NC = _INFO.num_cores
_NS = _INFO.num_subcores
_NW = _NC * _NS
_CHUNK = 128
_NBUF = 6


def _make_stage_b_kernel(e0, epw):
    nfull = epw // _CHUNK
    tail = epw - nfull * _CHUNK
    nouter = nfull // _NBUF
    nrem = nfull - nouter * _NBUF

    def body(p_hbm, q_hbm, s_hbm, e_hbm, out1_hbm, out2_hbm,
             idx_s, idx_e, *bufs):
        rows_s = bufs[0:_NBUF]
        rows_e = bufs[_NBUF:2 * _NBUF]
        sem_gs = bufs[2 * _NBUF:3 * _NBUF]
        sem_ge = bufs[3 * _NBUF:4 * _NBUF]
        sem_os = bufs[4 * _NBUF:5 * _NBUF]
        sem_oe = bufs[5 * _NBUF:6 * _NBUF]

        wid = lax.axis_index("s") * _NC + lax.axis_index("c")
        base = e0 + wid * epw
        obase = wid * epw

        def gather_pair(c, b):
            si = idx_s.at[pl.ds(c * _CHUNK, _CHUNK)]
            ei = idx_e.at[pl.ds(c * _CHUNK, _CHUNK)]
            pltpu.async_copy(p_hbm.at[si], rows_s[b], sem_gs[b])
            pltpu.async_copy(q_hbm.at[ei], rows_e[b], sem_ge[b])

        def wait_gather(c, b):
            si = idx_s.at[pl.ds(c * _CHUNK, _CHUNK)]
            ei = idx_e.at[pl.ds(c * _CHUNK, _CHUNK)]
            pltpu.make_async_copy(p_hbm.at[si], rows_s[b], sem_gs[b]).wait()
            pltpu.make_async_copy(q_hbm.at[ei], rows_e[b], sem_ge[b]).wait()

        def start_out(c, b):
            cb = obase + c * _CHUNK
            pltpu.async_copy(rows_s[b], out1_hbm.at[pl.ds(cb, _CHUNK)],
                             sem_os[b])
            pltpu.async_copy(rows_e[b], out2_hbm.at[pl.ds(cb, _CHUNK)],
                             sem_oe[b])

        def wait_out(c, b):
            cb = obase + c * _CHUNK
            pltpu.make_async_copy(
                rows_s[b], out1_hbm.at[pl.ds(cb, _CHUNK)], sem_os[b]).wait()
            pltpu.make_async_copy(
                rows_e[b], out2_hbm.at[pl.ds(cb, _CHUNK)], sem_oe[b]).wait()

        def emit(t, b, dyn):
            wait_gather(t, b)
            start_out(t, b)
            tg = t + 4
            bg = (b + 4) % _NBUF
            if dyn:
                @pl.when(tg < nfull)
                def _():
                    @pl.when(t >= 2)
                    def _():
                        wait_out(t - 2, bg)
                    gather_pair(tg, bg)
            else:
                if tg < nfull:
                    if t >= 2:
                        wait_out(t - 2, bg)
                    gather_pair(tg, bg)

        pltpu.sync_copy(s_hbm.at[pl.ds(base, epw)], idx_s)
        pltpu.sync_copy(e_hbm.at[pl.ds(base, epw)], idx_e)

        for b in range(4):
            gather_pair(jnp.int32(b), b)

        def outer(g, carry):
            for b in range(_NBUF):
                emit(g * _NBUF + b, b, True)
            return carry

        lax.fori_loop(0, nouter, outer, None)
        for k in range(nrem):
            t = nouter * _NBUF + k
            emit(t, t % _NBUF, False)

        for k in range(_NBUF):
            c = nfull - _NBUF + k
            wait_out(jnp.int32(c), c % _NBUF)

        if tail:
            tb = obase + nfull * _CHUNK
            si = idx_s.at[pl.ds(nfull * _CHUNK, tail)]
            ei = idx_e.at[pl.ds(nfull * _CHUNK, tail)]
            ts = rows_s[0].at[pl.ds(0, tail), :]
            te = rows_e[0].at[pl.ds(0, tail), :]
            pltpu.async_copy(p_hbm.at[si], ts, sem_gs[0])
            pltpu.async_copy(q_hbm.at[ei], te, sem_ge[0])
            pltpu.make_async_copy(p_hbm.at[si], ts, sem_gs[0]).wait()
            pltpu.make_async_copy(q_hbm.at[ei], te, sem_ge[0]).wait()
            pltpu.sync_copy(ts, out1_hbm.at[pl.ds(tb, tail)])
            pltpu.sync_copy(te, out2_hbm.at[pl.ds(tb, tail)])

    return body


def _stage_b(p_tab, q_tab, start, end, e0, epw):
    eh = epw * _NW
    fn = functools.partial(
        pl.kernel,
        mesh=plsc.VectorSubcoreMesh(core_axis_name="c", subcore_axis_name="s"),
        compiler_params=pltpu.CompilerParams(use_tc_tiling_on_sc=False),
        out_type=[
            jax.ShapeDtypeStruct((eh, H), jnp.float32),
            jax.ShapeDtypeStruct((eh, H), jnp.float32),
        ],
        scratch_types=[
            pltpu.VMEM((epw,), jnp.int32),
            pltpu.VMEM((epw,), jnp.int32),
        ]
        + [pltpu.VMEM((_CHUNK, H), jnp.float32) for _ in range(2 * _NBUF)]
        + [pltpu.SemaphoreType.DMA for _ in range(4 * _NBUF)],
    )(_make_stage_b_kernel(e0, epw))
    return fn(p_tab, q_tab, start, end)


def _stage_c_body(s_ref, e_ref, w2_ref, w3_ref, w4_ref, vraw_ref, out_ref):
    f32 = jnp.float32
    ii = lax.broadcasted_iota(jnp.int32, (128, 128), 0)
    jj = lax.broadcasted_iota(jnp.int32, (128, 128), 1)
    bd1 = jnp.where((ii // H) == (jj // H), 1.0, 0.0).astype(f32)
    si = lax.broadcasted_iota(jnp.int32, (128, H), 0)
    sj = lax.broadcasted_iota(jnp.int32, (128, H), 1)
    s_mat = jnp.where((si % H) == sj, 1.0, 0.0).astype(f32)
    t8i = lax.broadcasted_iota(jnp.int32, (H, 128), 0)
    t8j = lax.broadcasted_iota(jnp.int32, (H, 128), 1)
    t8m = jnp.where((t8j % H) == t8i, 1.0, 0.0).astype(f32)

    def bd(w):
        tile = jnp.dot(jnp.dot(s_mat, w, preferred_element_type=f32), t8m,
                       preferred_element_type=f32)
        return tile * bd1

    w2bd = bd(w2_ref[...])
    w3bd = bd(w3_ref[...])
    ci = lax.broadcasted_iota(jnp.int32, (128, GROUPS), 0)
    cg = lax.broadcasted_iota(jnp.int32, (128, GROUPS), 1)
    colmask = jnp.where((ci // H) == cg, 1.0, 0.0).astype(f32)
    w4col = jnp.dot(s_mat, w4_ref[...], preferred_element_type=f32)
    c4 = w4col * colmask
    vt = jnp.dot(vraw_ref[...], t8m, preferred_element_type=f32)

    def ln_relu(z, g, be):
        m = jnp.dot(z, bd1, preferred_element_type=f32) * (1.0 / H)
        zc = z - m
        v = jnp.dot(zc * zc, bd1, preferred_element_type=f32) * (1.0 / H)
        z = zc / jnp.sqrt(v + EPS) * g + be
        return jnp.maximum(z, 0.0)

    z = s_ref[...] + e_ref[...]
    z = ln_relu(z, vt[0:1, :], vt[1:2, :])
    z = jnp.dot(z, w2bd, preferred_element_type=f32) + vt[2:3, :]
    z = ln_relu(z, vt[3:4, :], vt[4:5, :])
    z = jnp.dot(z, w3bd, preferred_element_type=f32) + vt[5:6, :]
    z = ln_relu(z, vt[6:7, :], vt[7:8, :])
    out_ref[...] = (
        jnp.dot(z, c4, preferred_element_type=f32) + vt[8:9, 0:GROUPS]
    )


def _stage_c(zs, ze, w2, w3, w4, vraw, rh, grid):
    rb = rh // grid
    return pl.pallas_call(
        _stage_c_body,
        grid=(grid,),
        in_specs=[
            pl.BlockSpec((rb, 128), lambda i: (i, 0)),
            pl.BlockSpec((rb, 128), lambda i: (i, 0)),
            pl.BlockSpec((H, H), lambda i: (0, 0)),
            pl.BlockSpec((H, H), lambda i: (0, 0)),
            pl.BlockSpec((H, 1), lambda i: (0, 0)),
            pl.BlockSpec((9, H), lambda i: (0, 0)),
        ],
        out_specs=pl.BlockSpec((rb, GROUPS), lambda i: (i, 0)),
        out_shape=jax.ShapeDtypeStruct((rh, GROUPS), jnp.float32),
    )(zs, ze, w2, w3, w4, vraw)


def kernel(x, edge_index, W1, b1, g1, be1, W2, b2, g2, be2, W3, b3, g3, be3,
           W4, b4):
    bias = b1[None, :]
    vraw = jnp.stack([g1, be1, b2, g2, be2, b3, g3, be3,
                      jnp.full((H,), b4[0], jnp.float32)])

    start = edge_index[0].astype(jnp.int32)
    end = edge_index[1].astype(jnp.int32)

    p_tab, q_tab = _stage_a(x, W1, bias)

    epw1 = 5120
    e_half = epw1 * _NW
    epw2 = E // _NW - epw1
    e_rest = epw2 * _NW

    s1, t1 = _stage_b(p_tab, q_tab, start, end, 0, epw1)
    s2, t2 = _stage_b(p_tab, q_tab, start, end, e_half, epw2)

    r1 = e_half // GROUPS
    r2 = e_rest // GROUPS
    o1 = _stage_c(s1.reshape(r1, 128), t1.reshape(r1, 128),
                  W2, W3, W4, vraw, r1, 5)
    o2 = _stage_c(s2.reshape(r2, 128), t2.reshape(r2, 128),
                  W2, W3, W4, vraw, r2, 5)
    return jnp.concatenate([o1.reshape(e_half), o2.reshape(e_rest)])


# speedup vs baseline: 1.0377x; 1.0156x over previous
import functools

import jax
import jax.numpy as jnp
from jax import lax
from jax.experimental import pallas as pl
from jax.experimental.pallas import tpu as pltpu
from jax.experimental.pallas import tpu_sc as plsc

N = 10000
D = 128
E = 320000
H = 8
GROUPS = 16
R = E // GROUPS
EPS = 1e-5


def _stage_a_body(x_ref, w1_ref, bias_ref, p_ref, q_ref):
    x = x_ref[...]
    p_ref[...] = (
        jnp.dot(x, w1_ref[0:D, :], preferred_element_type=jnp.float32)
        + bias_ref[...]
    )
    q_ref[...] = jnp.dot(x, w1_ref[D:2 * D, :],
                         preferred_element_type=jnp.float32)


def _stage_a(x, w1, bias):
    return pl.pallas_call(
        _stage_a_body,
        out_shape=[
            jax.ShapeDtypeStruct((N, H), jnp.float32),
            jax.ShapeDtypeStruct((N, H), jnp.float32),
        ],
    )(x, w1, bias)


_INFO = plsc.get_sparse_core_info()
_NC = _INFO.num_cores
_NS = _INFO.num_subcores
_NW = _NC * _NS
_CHUNK = 128
_NBUF = 6


def _make_stage_b_kernel(e0, epw):
    nfull = epw // _CHUNK
    tail = epw - nfull * _CHUNK
    nouter = nfull // _NBUF
    nrem = nfull - nouter * _NBUF

    def body(p_hbm, q_hbm, s_hbm, e_hbm, out1_hbm, out2_hbm,
             idx_s, idx_e, *bufs):
        rows_s = bufs[0:_NBUF]
        rows_e = bufs[_NBUF:2 * _NBUF]
        sem_gs = bufs[2 * _NBUF:3 * _NBUF]
        sem_ge = bufs[3 * _NBUF:4 * _NBUF]
        sem_os = bufs[4 * _NBUF:5 * _NBUF]
        sem_oe = bufs[5 * _NBUF:6 * _NBUF]

        wid = lax.axis_index("s") * _NC + lax.axis_index("c")
        base = e0 + wid * epw
        obase = wid * epw

        def gather_pair(c, b):
            si = idx_s.at[pl.ds(c * _CHUNK, _CHUNK)]
            ei = idx_e.at[pl.ds(c * _CHUNK, _CHUNK)]
            pltpu.async_copy(p_hbm.at[si], rows_s[b], sem_gs[b])
            pltpu.async_copy(q_hbm.at[ei], rows_e[b], sem_ge[b])

        def wait_gather(c, b):
            si = idx_s.at[pl.ds(c * _CHUNK, _CHUNK)]
            ei = idx_e.at[pl.ds(c * _CHUNK, _CHUNK)]
            pltpu.make_async_copy(p_hbm.at[si], rows_s[b], sem_gs[b]).wait()
            pltpu.make_async_copy(q_hbm.at[ei], rows_e[b], sem_ge[b]).wait()

        def start_out(c, b):
            cb = obase + c * _CHUNK
            pltpu.async_copy(rows_s[b], out1_hbm.at[pl.ds(cb, _CHUNK)],
                             sem_os[b])
            pltpu.async_copy(rows_e[b], out2_hbm.at[pl.ds(cb, _CHUNK)],
                             sem_oe[b])

        def wait_out(c, b):
            cb = obase + c * _CHUNK
            pltpu.make_async_copy(
                rows_s[b], out1_hbm.at[pl.ds(cb, _CHUNK)], sem_os[b]).wait()
            pltpu.make_async_copy(
                rows_e[b], out2_hbm.at[pl.ds(cb, _CHUNK)], sem_oe[b]).wait()

        def emit(t, b, dyn):
            wait_gather(t, b)
            start_out(t, b)
            tg = t + 4
            bg = (b + 4) % _NBUF
            if dyn:
                @pl.when(tg < nfull)
                def _():
                    @pl.when(t >= 2)
                    def _():
                        wait_out(t - 2, bg)
                    gather_pair(tg, bg)
            else:
                if tg < nfull:
                    if t >= 2:
                        wait_out(t - 2, bg)
                    gather_pair(tg, bg)

        ic1 = pltpu.async_copy(s_hbm.at[pl.ds(base, epw)], idx_s, sem_gs[0])
        ic2 = pltpu.async_copy(e_hbm.at[pl.ds(base, epw)], idx_e, sem_ge[0])
        ic1.wait()
        ic2.wait()

        for b in range(4):
            gather_pair(jnp.int32(b), b)

        def outer(g, carry):
            for b in range(_NBUF):
                emit(g * _NBUF + b, b, True)
            return carry

        lax.fori_loop(0, nouter, outer, None)
        for k in range(nrem):
            t = nouter * _NBUF + k
            emit(t, t % _NBUF, False)

        for k in range(_NBUF):
            c = nfull - _NBUF + k
            wait_out(jnp.int32(c), c % _NBUF)

        if tail:
            tb = obase + nfull * _CHUNK
            si = idx_s.at[pl.ds(nfull * _CHUNK, tail)]
            ei = idx_e.at[pl.ds(nfull * _CHUNK, tail)]
            ts = rows_s[0].at[pl.ds(0, tail), :]
            te = rows_e[0].at[pl.ds(0, tail), :]
            pltpu.async_copy(p_hbm.at[si], ts, sem_gs[0])
            pltpu.async_copy(q_hbm.at[ei], te, sem_ge[0])
            pltpu.make_async_copy(p_hbm.at[si], ts, sem_gs[0]).wait()
            pltpu.make_async_copy(q_hbm.at[ei], te, sem_ge[0]).wait()
            pltpu.sync_copy(ts, out1_hbm.at[pl.ds(tb, tail)])
            pltpu.sync_copy(te, out2_hbm.at[pl.ds(tb, tail)])

    return body


def _stage_b(p_tab, q_tab, start, end, e0, epw):
    eh = epw * _NW
    fn = functools.partial(
        pl.kernel,
        mesh=plsc.VectorSubcoreMesh(core_axis_name="c", subcore_axis_name="s"),
        compiler_params=pltpu.CompilerParams(use_tc_tiling_on_sc=False),
        out_type=[
            jax.ShapeDtypeStruct((eh, H), jnp.float32),
            jax.ShapeDtypeStruct((eh, H), jnp.float32),
        ],
        scratch_types=[
            pltpu.VMEM((epw,), jnp.int32),
            pltpu.VMEM((epw,), jnp.int32),
        ]
        + [pltpu.VMEM((_CHUNK, H), jnp.float32) for _ in range(2 * _NBUF)]
        + [pltpu.SemaphoreType.DMA for _ in range(4 * _NBUF)],
    )(_make_stage_b_kernel(e0, epw))
    return fn(p_tab, q_tab, start, end)


def _stage_c_body(s_ref, e_ref, w2_ref, w3_ref, w4_ref, vraw_ref, out_ref):
    f32 = jnp.float32
    ii = lax.broadcasted_iota(jnp.int32, (128, 128), 0)
    jj = lax.broadcasted_iota(jnp.int32, (128, 128), 1)
    bd1 = jnp.where((ii // H) == (jj // H), 1.0, 0.0).astype(f32)
    si = lax.broadcasted_iota(jnp.int32, (128, H), 0)
    sj = lax.broadcasted_iota(jnp.int32, (128, H), 1)
    s_mat = jnp.where((si % H) == sj, 1.0, 0.0).astype(f32)
    t8i = lax.broadcasted_iota(jnp.int32, (H, 128), 0)
    t8j = lax.broadcasted_iota(jnp.int32, (H, 128), 1)
    t8m = jnp.where((t8j % H) == t8i, 1.0, 0.0).astype(f32)

    def bd(w):
        tile = jnp.dot(jnp.dot(s_mat, w, preferred_element_type=f32), t8m,
                       preferred_element_type=f32)
        return tile * bd1

    w2bd = bd(w2_ref[...])
    w3bd = bd(w3_ref[...])
    ci = lax.broadcasted_iota(jnp.int32, (128, GROUPS), 0)
    cg = lax.broadcasted_iota(jnp.int32, (128, GROUPS), 1)
    colmask = jnp.where((ci // H) == cg, 1.0, 0.0).astype(f32)
    w4col = jnp.dot(s_mat, w4_ref[...], preferred_element_type=f32)
    c4 = w4col * colmask
    vt = jnp.dot(vraw_ref[...], t8m, preferred_element_type=f32)

    def ln_relu(z, g, be):
        m = jnp.dot(z, bd1, preferred_element_type=f32) * (1.0 / H)
        zc = z - m
        v = jnp.dot(zc * zc, bd1, preferred_element_type=f32) * (1.0 / H)
        z = zc / jnp.sqrt(v + EPS) * g + be
        return jnp.maximum(z, 0.0)

    z = s_ref[...] + e_ref[...]
    z = ln_relu(z, vt[0:1, :], vt[1:2, :])
    z = jnp.dot(z, w2bd, preferred_element_type=f32) + vt[2:3, :]
    z = ln_relu(z, vt[3:4, :], vt[4:5, :])
    z = jnp.dot(z, w3bd, preferred_element_type=f32) + vt[5:6, :]
    z = ln_relu(z, vt[6:7, :], vt[7:8, :])
    out_ref[...] = (
        jnp.dot(z, c4, preferred_element_type=f32) + vt[8:9, 0:GROUPS]
    )


def _stage_c(zs, ze, w2, w3, w4, vraw, rh, grid):
    rb = rh // grid
    return pl.pallas_call(
        _stage_c_body,
        grid=(grid,),
        in_specs=[
            pl.BlockSpec((rb, 128), lambda i: (i, 0)),
            pl.BlockSpec((rb, 128), lambda i: (i, 0)),
            pl.BlockSpec((H, H), lambda i: (0, 0)),
            pl.BlockSpec((H, H), lambda i: (0, 0)),
            pl.BlockSpec((H, 1), lambda i: (0, 0)),
            pl.BlockSpec((9, H), lambda i: (0, 0)),
        ],
        out_specs=pl.BlockSpec((rb, GROUPS), lambda i: (i, 0)),
        out_shape=jax.ShapeDtypeStruct((rh, GROUPS), jnp.float32),
    )(zs, ze, w2, w3, w4, vraw)


def kernel(x, edge_index, W1, b1, g1, be1, W2, b2, g2, be2, W3, b3, g3, be3,
           W4, b4):
    bias = b1[None, :]
    vraw = jnp.stack([g1, be1, b2, g2, be2, b3, g3, be3,
                      jnp.full((H,), b4[0], jnp.float32)])

    start = edge_index[0].astype(jnp.int32)
    end = edge_index[1].astype(jnp.int32)

    p_tab, q_tab = _stage_a(x, W1, bias)

    epw1 = 5120
    e_half = epw1 * _NW
    epw2 = E // _NW - epw1
    e_rest = epw2 * _NW

    s1, t1 = _stage_b(p_tab, q_tab, start, end, 0, epw1)
    s2, t2 = _stage_b(p_tab, q_tab, start, end, e_half, epw2)

    r1 = e_half // GROUPS
    r2 = e_rest // GROUPS
    o1 = _stage_c(s1.reshape(r1, 128), t1.reshape(r1, 128),
                  W2, W3, W4, vraw, r1, 5)
    o2 = _stage_c(s2.reshape(r2, 128), t2.reshape(r2, 128),
                  W2, W3, W4, vraw, r2, 5)
    return jnp.concatenate([o1.reshape(e_half), o2.reshape(e_rest)])


# speedup vs baseline: 1.1055x; 1.0653x over previous
import functools

import jax
import jax.numpy as jnp
from jax import lax
from jax.experimental import pallas as pl
from jax.experimental.pallas import tpu as pltpu
from jax.experimental.pallas import tpu_sc as plsc

N = 10000
D = 128
E = 320000
H = 8
GROUPS = 16
R = E // GROUPS
EPS = 1e-5


_PACK = 16
_NPR = N // _PACK


def _stage_a_body(xr_ref, w1_ref, bias_ref, p_ref, q_ref):
    f32 = jnp.float32
    bi = lax.broadcasted_iota(jnp.int32, (_PACK * D, D), 0)
    bk = lax.broadcasted_iota(jnp.int32, (_PACK * D, D), 1)
    lift = jnp.where((bi % D) == bk, 1.0, 0.0).astype(f32)
    wi = lax.broadcasted_iota(jnp.int32, (_PACK * D, 128), 0)
    wj = lax.broadcasted_iota(jnp.int32, (_PACK * D, 128), 1)
    blockmask = jnp.where((wi // D) == (wj // H), 1.0, 0.0).astype(f32)
    ti = lax.broadcasted_iota(jnp.int32, (H, 128), 0)
    tj = lax.broadcasted_iota(jnp.int32, (H, 128), 1)
    t8m = jnp.where((tj % H) == ti, 1.0, 0.0).astype(f32)

    def wbig(w):
        tile = jnp.dot(jnp.dot(lift, w, preferred_element_type=f32), t8m,
                       preferred_element_type=f32)
        return tile * blockmask

    xr = xr_ref[...]
    bias_t = jnp.dot(bias_ref[...], t8m, preferred_element_type=f32)
    p_ref[...] = (
        jnp.dot(xr, wbig(w1_ref[0:D, :]), preferred_element_type=f32)
        + bias_t
    )
    q_ref[...] = jnp.dot(xr, wbig(w1_ref[D:2 * D, :]),
                         preferred_element_type=f32)


def _stage_a(x, w1, bias):
    return pl.pallas_call(
        _stage_a_body,
        out_shape=[
            jax.ShapeDtypeStruct((_NPR, 128), jnp.float32),
            jax.ShapeDtypeStruct((_NPR, 128), jnp.float32),
        ],
    )(x.reshape(_NPR, _PACK * D), w1, bias)


_INFO = plsc.get_sparse_core_info()
_NC = _INFO.num_cores
_NS = _INFO.num_subcores
_NW = _NC * _NS
_CHUNK = 128
_NBUF = 6


def _make_stage_b_kernel(e0, epw):
    nfull = epw // _CHUNK
    tail = epw - nfull * _CHUNK
    nouter = nfull // _NBUF
    nrem = nfull - nouter * _NBUF

    def body(p_hbm, q_hbm, s_hbm, e_hbm, out1_hbm, out2_hbm,
             idx_s, idx_e, *bufs):
        rows_s = bufs[0:_NBUF]
        rows_e = bufs[_NBUF:2 * _NBUF]
        sem_gs = bufs[2 * _NBUF:3 * _NBUF]
        sem_ge = bufs[3 * _NBUF:4 * _NBUF]
        sem_os = bufs[4 * _NBUF:5 * _NBUF]
        sem_oe = bufs[5 * _NBUF:6 * _NBUF]

        wid = lax.axis_index("s") * _NC + lax.axis_index("c")
        base = e0 + wid * epw
        obase = wid * epw

        def gather_pair(c, b):
            si = idx_s.at[pl.ds(c * _CHUNK, _CHUNK)]
            ei = idx_e.at[pl.ds(c * _CHUNK, _CHUNK)]
            pltpu.async_copy(p_hbm.at[si], rows_s[b], sem_gs[b])
            pltpu.async_copy(q_hbm.at[ei], rows_e[b], sem_ge[b])

        def wait_gather(c, b):
            si = idx_s.at[pl.ds(c * _CHUNK, _CHUNK)]
            ei = idx_e.at[pl.ds(c * _CHUNK, _CHUNK)]
            pltpu.make_async_copy(p_hbm.at[si], rows_s[b], sem_gs[b]).wait()
            pltpu.make_async_copy(q_hbm.at[ei], rows_e[b], sem_ge[b]).wait()

        def start_out(c, b):
            cb = obase + c * _CHUNK
            pltpu.async_copy(rows_s[b], out1_hbm.at[pl.ds(cb, _CHUNK)],
                             sem_os[b])
            pltpu.async_copy(rows_e[b], out2_hbm.at[pl.ds(cb, _CHUNK)],
                             sem_oe[b])

        def wait_out(c, b):
            cb = obase + c * _CHUNK
            pltpu.make_async_copy(
                rows_s[b], out1_hbm.at[pl.ds(cb, _CHUNK)], sem_os[b]).wait()
            pltpu.make_async_copy(
                rows_e[b], out2_hbm.at[pl.ds(cb, _CHUNK)], sem_oe[b]).wait()

        def emit(t, b, dyn):
            wait_gather(t, b)
            start_out(t, b)
            tg = t + 4
            bg = (b + 4) % _NBUF
            if dyn:
                @pl.when(tg < nfull)
                def _():
                    @pl.when(t >= 2)
                    def _():
                        wait_out(t - 2, bg)
                    gather_pair(tg, bg)
            else:
                if tg < nfull:
                    if t >= 2:
                        wait_out(t - 2, bg)
                    gather_pair(tg, bg)

        ic1 = pltpu.async_copy(s_hbm.at[pl.ds(base, epw)], idx_s, sem_gs[0])
        ic2 = pltpu.async_copy(e_hbm.at[pl.ds(base, epw)], idx_e, sem_ge[0])
        ic1.wait()
        ic2.wait()

        for b in range(4):
            gather_pair(jnp.int32(b), b)

        def outer(g, carry):
            for b in range(_NBUF):
                emit(g * _NBUF + b, b, True)
            return carry

        lax.fori_loop(0, nouter, outer, None)
        for k in range(nrem):
            t = nouter * _NBUF + k
            emit(t, t % _NBUF, False)

        for k in range(_NBUF):
            c = nfull - _NBUF + k
            wait_out(jnp.int32(c), c % _NBUF)

        if tail:
            tb = obase + nfull * _CHUNK
            si = idx_s.at[pl.ds(nfull * _CHUNK, tail)]
            ei = idx_e.at[pl.ds(nfull * _CHUNK, tail)]
            ts = rows_s[0].at[pl.ds(0, tail), :]
            te = rows_e[0].at[pl.ds(0, tail), :]
            pltpu.async_copy(p_hbm.at[si], ts, sem_gs[0])
            pltpu.async_copy(q_hbm.at[ei], te, sem_ge[0])
            pltpu.make_async_copy(p_hbm.at[si], ts, sem_gs[0]).wait()
            pltpu.make_async_copy(q_hbm.at[ei], te, sem_ge[0]).wait()
            pltpu.sync_copy(ts, out1_hbm.at[pl.ds(tb, tail)])
            pltpu.sync_copy(te, out2_hbm.at[pl.ds(tb, tail)])

    return body


def _stage_b(p_tab, q_tab, start, end, e0, epw):
    eh = epw * _NW
    fn = functools.partial(
        pl.kernel,
        mesh=plsc.VectorSubcoreMesh(core_axis_name="c", subcore_axis_name="s"),
        compiler_params=pltpu.CompilerParams(use_tc_tiling_on_sc=False),
        out_type=[
            jax.ShapeDtypeStruct((eh, H), jnp.float32),
            jax.ShapeDtypeStruct((eh, H), jnp.float32),
        ],
        scratch_types=[
            pltpu.VMEM((epw,), jnp.int32),
            pltpu.VMEM((epw,), jnp.int32),
        ]
        + [pltpu.VMEM((_CHUNK, H), jnp.float32) for _ in range(2 * _NBUF)]
        + [pltpu.SemaphoreType.DMA for _ in range(4 * _NBUF)],
    )(_make_stage_b_kernel(e0, epw))
    return fn(p_tab, q_tab, start, end)


def _stage_c_body(s_ref, e_ref, w2_ref, w3_ref, w4_ref, vraw_ref, out_ref):
    f32 = jnp.float32
    ii = lax.broadcasted_iota(jnp.int32, (128, 128), 0)
    jj = lax.broadcasted_iota(jnp.int32, (128, 128), 1)
    bd1 = jnp.where((ii // H) == (jj // H), 1.0, 0.0).astype(f32)
    si = lax.broadcasted_iota(jnp.int32, (128, H), 0)
    sj = lax.broadcasted_iota(jnp.int32, (128, H), 1)
    s_mat = jnp.where((si % H) == sj, 1.0, 0.0).astype(f32)
    t8i = lax.broadcasted_iota(jnp.int32, (H, 128), 0)
    t8j = lax.broadcasted_iota(jnp.int32, (H, 128), 1)
    t8m = jnp.where((t8j % H) == t8i, 1.0, 0.0).astype(f32)

    def bd(w):
        tile = jnp.dot(jnp.dot(s_mat, w, preferred_element_type=f32), t8m,
                       preferred_element_type=f32)
        return tile * bd1

    w2bd = bd(w2_ref[...])
    w3bd = bd(w3_ref[...])
    ci = lax.broadcasted_iota(jnp.int32, (128, GROUPS), 0)
    cg = lax.broadcasted_iota(jnp.int32, (128, GROUPS), 1)
    colmask = jnp.where((ci // H) == cg, 1.0, 0.0).astype(f32)
    w4col = jnp.dot(s_mat, w4_ref[...], preferred_element_type=f32)
    c4 = w4col * colmask
    vt = jnp.dot(vraw_ref[...], t8m, preferred_element_type=f32)

    def ln_relu(z, g, be):
        m = jnp.dot(z, bd1, preferred_element_type=f32) * (1.0 / H)
        zc = z - m
        v = jnp.dot(zc * zc, bd1, preferred_element_type=f32) * (1.0 / H)
        z = zc / jnp.sqrt(v + EPS) * g + be
        return jnp.maximum(z, 0.0)

    z = s_ref[...] + e_ref[...]
    z = ln_relu(z, vt[0:1, :], vt[1:2, :])
    z = jnp.dot(z, w2bd, preferred_element_type=f32) + vt[2:3, :]
    z = ln_relu(z, vt[3:4, :], vt[4:5, :])
    z = jnp.dot(z, w3bd, preferred_element_type=f32) + vt[5:6, :]
    z = ln_relu(z, vt[6:7, :], vt[7:8, :])
    out_ref[...] = (
        jnp.dot(z, c4, preferred_element_type=f32) + vt[8:9, 0:GROUPS]
    )


def _stage_c(zs, ze, w2, w3, w4, vraw, rh, grid):
    rb = rh // grid
    return pl.pallas_call(
        _stage_c_body,
        grid=(grid,),
        in_specs=[
            pl.BlockSpec((rb, 128), lambda i: (i, 0)),
            pl.BlockSpec((rb, 128), lambda i: (i, 0)),
            pl.BlockSpec((H, H), lambda i: (0, 0)),
            pl.BlockSpec((H, H), lambda i: (0, 0)),
            pl.BlockSpec((H, 1), lambda i: (0, 0)),
            pl.BlockSpec((9, H), lambda i: (0, 0)),
        ],
        out_specs=pl.BlockSpec((rb, GROUPS), lambda i: (i, 0)),
        out_shape=jax.ShapeDtypeStruct((rh, GROUPS), jnp.float32),
    )(zs, ze, w2, w3, w4, vraw)


def kernel(x, edge_index, W1, b1, g1, be1, W2, b2, g2, be2, W3, b3, g3, be3,
           W4, b4):
    bias = b1[None, :]
    vraw = jnp.stack([g1, be1, b2, g2, be2, b3, g3, be3,
                      jnp.full((H,), b4[0], jnp.float32)])

    start = edge_index[0].astype(jnp.int32)
    end = edge_index[1].astype(jnp.int32)

    p_pk, q_pk = _stage_a(x, W1, bias)
    p_tab = p_pk.reshape(N, H)
    q_tab = q_pk.reshape(N, H)

    epw1 = 5120
    e_half = epw1 * _NW
    epw2 = E // _NW - epw1
    e_rest = epw2 * _NW

    s1, t1 = _stage_b(p_tab, q_tab, start, end, 0, epw1)
    s2, t2 = _stage_b(p_tab, q_tab, start, end, e_half, epw2)

    r1 = e_half // GROUPS
    r2 = e_rest // GROUPS
    o1 = _stage_c(s1.reshape(r1, 128), t1.reshape(r1, 128),
                  W2, W3, W4, vraw, r1, 5)
    o2 = _stage_c(s2.reshape(r2, 128), t2.reshape(r2, 128),
                  W2, W3, W4, vraw, r2, 5)
    return jnp.concatenate([o1.reshape(e_half), o2.reshape(e_rest)])
